# Initial kernel scaffold; baseline (speedup 1.0000x reference)
import jax
import jax.numpy as jnp
from jax.experimental import pallas as pl


def kernel(features, structures, w1, b1, w2, b2, w3, b3, sw1_top, sw1_bot, sb1, sw2t, sb2c, sw3c, sb3):
    raise NotImplementedError("write your pallas kernel here")


# speedup vs baseline: 1.1993x; 1.1993x over previous
import numpy as np

import jax
import jax.numpy as jnp
from jax.experimental import pallas as pl
from jax.experimental.pallas import tpu as pltpu

_BF16 = jnp.bfloat16
_VMEM_LIMIT = 48 * 1024 * 1024
_CW = 0.1
_ROW_TILE = 512
_PAIR_TI = 16

_HYDRO_T = {'A': 1.8, 'C': 2.5, 'D': -3.5, 'E': -3.5, 'F': 2.8, 'G': -0.4,
            'H': -3.2, 'I': 4.5, 'K': -3.9, 'L': 3.8, 'M': 1.9, 'N': -3.5,
            'P': -1.6, 'Q': -3.5, 'R': -4.5, 'S': -0.8, 'T': -0.7,
            'V': 4.2, 'W': -0.9, 'Y': -1.3}
_CHARGE_T = {'A': 0, 'C': 0, 'D': -1, 'E': -1, 'F': 0, 'G': 0, 'H': 0.5,
             'I': 0, 'K': 1, 'L': 0, 'M': 0, 'N': 0, 'P': 0, 'Q': 0,
             'R': 1, 'S': 0, 'T': 0, 'V': 0, 'W': 0, 'Y': 0}
_DONOR_T = {'A': 0, 'C': 0, 'D': 0, 'E': 0, 'F': 0, 'G': 0, 'H': 1, 'I': 0,
            'K': 1, 'L': 0, 'M': 0, 'N': 1, 'P': 0, 'Q': 1, 'R': 2, 'S': 1,
            'T': 1, 'V': 0, 'W': 1, 'Y': 1}
_ACCEPT_T = {'A': 0, 'C': 0, 'D': 2, 'E': 2, 'F': 0, 'G': 0, 'H': 1,
             'I': 0, 'K': 0, 'L': 0, 'M': 1, 'N': 1, 'P': 0, 'Q': 1,
             'R': 0, 'S': 1, 'T': 1, 'V': 0, 'W': 0, 'Y': 1}
_AA = "ACDEFGHIKLMNPQRSTVWY"


def _targets(B, S):
    hyd = np.array([(_HYDRO_T[a] + 5.0) / 10.0 for a in _AA], np.float32)
    cls = np.array([0 if _CHARGE_T[a] > 0 else (1 if _CHARGE_T[a] < 0 else 2)
                    for a in _AA], np.int32)
    don = np.array([_DONOR_T[a] for a in _AA], np.float32)
    acc = np.array([_ACCEPT_T[a] for a in _AA], np.float32)
    idx = (np.arange(B, dtype=np.int64)[:, None] * S
           + np.arange(S, dtype=np.int64)[None, :]) % 20
    t_hydro = hyd[idx][..., None]
    labels = cls[idx].reshape(-1)
    t_hbond = np.stack([don[idx], acc[idx]], axis=-1)
    return jnp.asarray(t_hydro), jnp.asarray(labels), jnp.asarray(t_hbond)


def _row_kernel(x_ref, w1_ref, b1_ref, w2_ref, b2_ref, w3_ref, b3_ref,
                pa_w_ref, pb_w_ref, pb_b_ref,
                props_ref, feat_ref, pa_ref, pb_ref):
    x = x_ref[...]
    xb = x.astype(_BF16)
    h1 = jnp.maximum(
        jnp.dot(xb, w1_ref[...], preferred_element_type=jnp.float32)
        + b1_ref[...], 0.0)
    h2 = jnp.maximum(
        jnp.dot(h1.astype(_BF16), w2_ref[...],
                preferred_element_type=jnp.float32) + b2_ref[...], 0.0)
    z = jnp.dot(h2.astype(_BF16), w3_ref[...],
                preferred_element_type=jnp.float32) + b3_ref[...]

    hydro = jax.nn.sigmoid(z[:, 0:1])
    cz = z[:, 1:4]
    ce = jnp.exp(cz - jnp.max(cz, axis=1, keepdims=True))
    cp = ce / jnp.sum(ce, axis=1, keepdims=True)
    hb = z[:, 4:6]
    props_ref[...] = jnp.concatenate([hydro, cp, hb, z[:, 6:8]], axis=1)

    mod = ((1.0 + _CW * (hydro - 0.5))
           * (1.0 + _CW * (cp[:, 0:1] - cp[:, 1:2]))
           * (1.0 + _CW * (0.5 * (hb[:, 0:1] + hb[:, 1:2]) - 0.5)))
    feat_ref[...] = x * mod

    pa_ref[...] = jnp.dot(x, pa_w_ref[...], preferred_element_type=jnp.float32)
    pb_ref[...] = (jnp.dot(x, pb_w_ref[...], preferred_element_type=jnp.float32)
                   + pb_b_ref[...])


def _pair_kernel(pa_ref, pb_ref, don_ref, acc_ref, st_ref, stt_ref, stn_ref,
                 w2_ref, b2_ref, w3_ref, b3_ref, sp_ref, hb_ref):
    ti, hdim = pa_ref.shape
    spn = pb_ref.shape[0]

    h1 = jnp.maximum(pa_ref[...][:, None, :] + pb_ref[...][None, :, :], 0.0)
    h1f = h1.reshape(ti * spn, hdim).astype(_BF16)
    h2 = jnp.maximum(
        jnp.dot(h1f, w2_ref[...], preferred_element_type=jnp.float32)
        + b2_ref[...], 0.0)
    zlin = jnp.dot(h2, w3_ref[...],
                   preferred_element_type=jnp.float32) + b3_ref[0, 0]
    p = jax.nn.sigmoid(zlin.reshape(ti, spn))

    si = st_ref[...]
    si_n = jnp.sum(si * si, axis=1, keepdims=True)
    dots = jnp.dot(si, stt_ref[...], preferred_element_type=jnp.float32)
    d2 = si_n + stn_ref[...] - 2.0 * dots

    contact = d2 < 64.0
    hmask = jnp.logical_and(d2 > 6.25, d2 < 12.25)

    log_p = jnp.maximum(jnp.log(p), -100.0)
    log_1mp = jnp.maximum(jnp.log(1.0 - p), -100.0)
    term = jnp.where(contact, -log_p, -log_1mp)
    rows = (pl.program_id(1) * ti
            + jax.lax.broadcasted_iota(jnp.int32, (ti, spn), 0))
    cols = jax.lax.broadcasted_iota(jnp.int32, (ti, spn), 1)
    sp_sum = jnp.sum(jnp.where(cols > rows, 2.0 * term, 0.0))

    lgt = don_ref[...] * acc_ref[...]
    y = hmask.astype(jnp.float32)
    hb_sum = jnp.sum(jnp.maximum(lgt, 0.0) - lgt * y
                     + jnp.log1p(jnp.exp(-jnp.abs(lgt))))

    sp_ref[...] = jnp.broadcast_to(sp_sum, (1, 8, 128))
    hb_ref[...] = jnp.broadcast_to(hb_sum, (1, 8, 128))


def kernel(features, structures, w1, b1, w2, b2, w3, b3,
           sw1_top, sw1_bot, sb1, sw2t, sb2c, sw3c, sb3):
    B, S, D = features.shape
    H = sw1_top.shape[1]
    N = B * S
    x2d = features.reshape(N, D).astype(jnp.float32)

    tr = _ROW_TILE
    grid1 = (N // tr,)
    w_full = lambda w: pl.BlockSpec(w.shape, lambda i: (0,) * w.ndim)
    in_specs1 = ([pl.BlockSpec((tr, D), lambda i: (i, 0))]
                 + [w_full(w) for w in (w1, b1, w2, b2, w3, b3,
                                        sw1_top, sw1_bot, sb1)])
    out_specs1 = [pl.BlockSpec((tr, 8), lambda i: (i, 0)),
                  pl.BlockSpec((tr, D), lambda i: (i, 0)),
                  pl.BlockSpec((tr, H), lambda i: (i, 0)),
                  pl.BlockSpec((tr, H), lambda i: (i, 0))]
    out_shape1 = (jax.ShapeDtypeStruct((N, 8), jnp.float32),
                  jax.ShapeDtypeStruct((N, D), jnp.float32),
                  jax.ShapeDtypeStruct((N, H), jnp.float32),
                  jax.ShapeDtypeStruct((N, H), jnp.float32))
    props, feat2d, pa, pb = pl.pallas_call(
        _row_kernel, grid=grid1, in_specs=in_specs1, out_specs=out_specs1,
        out_shape=out_shape1,
        compiler_params=pltpu.CompilerParams(
            dimension_semantics=("parallel",),
            vmem_limit_bytes=_VMEM_LIMIT),
    )(x2d, w1, b1, w2, b2, w3, b3, sw1_top, sw1_bot, sb1)

    constrained = feat2d.reshape(B, S, D)
    props3 = props.reshape(B, S, 8)
    hydro = props3[:, :, 0:1]
    charge = props[:, 1:4]
    hbond = props3[:, :, 4:6]

    ti = _PAIR_TI
    pa3 = pa.reshape(B, S, H)
    pb3 = pb.reshape(B, S, H)
    don = props3[:, :, 4:5]
    acc = props3[:, :, 5][:, None, :]
    st = jnp.pad(structures.astype(jnp.float32), ((0, 0), (0, 0), (0, 5)))
    stt = jnp.swapaxes(st, 1, 2)
    stn = jnp.sum(st * st, axis=2)[:, None, :]
    w2p = sw2t.T
    b2r = sb2c.T

    grid2 = (B, S // ti)
    in_specs2 = [
        pl.BlockSpec((None, ti, H), lambda b, i: (b, i, 0)),
        pl.BlockSpec((None, S, H), lambda b, i: (b, 0, 0)),
        pl.BlockSpec((None, ti, 1), lambda b, i: (b, i, 0)),
        pl.BlockSpec((None, 1, S), lambda b, i: (b, 0, 0)),
        pl.BlockSpec((None, ti, 8), lambda b, i: (b, i, 0)),
        pl.BlockSpec((None, 8, S), lambda b, i: (b, 0, 0)),
        pl.BlockSpec((None, 1, S), lambda b, i: (b, 0, 0)),
        pl.BlockSpec(w2p.shape, lambda b, i: (0, 0)),
        pl.BlockSpec(b2r.shape, lambda b, i: (0, 0)),
        pl.BlockSpec(sw3c.shape, lambda b, i: (0, 0)),
        pl.BlockSpec(sb3.shape, lambda b, i: (0, 0)),
    ]
    out_specs2 = [pl.BlockSpec((None, 1, 8, 128), lambda b, i: (b, i, 0, 0)),
                  pl.BlockSpec((None, 1, 8, 128), lambda b, i: (b, i, 0, 0))]
    out_shape2 = (jax.ShapeDtypeStruct((B, S // ti, 8, 128), jnp.float32),
                  jax.ShapeDtypeStruct((B, S // ti, 8, 128), jnp.float32))
    sp_parts, hb_parts = pl.pallas_call(
        _pair_kernel, grid=grid2, in_specs=in_specs2, out_specs=out_specs2,
        out_shape=out_shape2,
        compiler_params=pltpu.CompilerParams(
            dimension_semantics=("parallel", "parallel"),
            vmem_limit_bytes=_VMEM_LIMIT),
    )(pa3, pb3, don, acc, st, stt, stn, w2p, b2r, sw3c, sb3)

    t_hydro, labels, t_hbond = _targets(B, S)
    losses = {}
    losses['hydrophobic'] = jnp.mean((hydro.reshape(B, S, 1) - t_hydro) ** 2)

    lse = jax.scipy.special.logsumexp(charge, axis=-1)
    picked = jnp.take_along_axis(charge, labels[:, None], axis=-1)[:, 0]
    losses['charge'] = jnp.mean(lse - picked)

    n_pairs = float(B * S * S)
    hb_bce = jnp.sum(hb_parts[:, :, 0, 0]) / n_pairs
    losses['hbond'] = jnp.mean((hbond - t_hbond) ** 2) + 0.5 * hb_bce

    losses['spatial'] = (jnp.sum(sp_parts[:, :, 0, 0]) + 100.0 * B * S) / n_pairs

    return constrained, losses


# speedup vs baseline: 1.3795x; 1.1502x over previous
import numpy as np

import jax
import jax.numpy as jnp
from jax.experimental import pallas as pl
from jax.experimental.pallas import tpu as pltpu

_BF16 = jnp.bfloat16
_VMEM_LIMIT = 48 * 1024 * 1024
_CW = 0.1
_ROW_TILE = 512
_PAIR_TI = 32

_HYDRO_T = {'A': 1.8, 'C': 2.5, 'D': -3.5, 'E': -3.5, 'F': 2.8, 'G': -0.4,
            'H': -3.2, 'I': 4.5, 'K': -3.9, 'L': 3.8, 'M': 1.9, 'N': -3.5,
            'P': -1.6, 'Q': -3.5, 'R': -4.5, 'S': -0.8, 'T': -0.7,
            'V': 4.2, 'W': -0.9, 'Y': -1.3}
_CHARGE_T = {'A': 0, 'C': 0, 'D': -1, 'E': -1, 'F': 0, 'G': 0, 'H': 0.5,
             'I': 0, 'K': 1, 'L': 0, 'M': 0, 'N': 0, 'P': 0, 'Q': 0,
             'R': 1, 'S': 0, 'T': 0, 'V': 0, 'W': 0, 'Y': 0}
_DONOR_T = {'A': 0, 'C': 0, 'D': 0, 'E': 0, 'F': 0, 'G': 0, 'H': 1, 'I': 0,
            'K': 1, 'L': 0, 'M': 0, 'N': 1, 'P': 0, 'Q': 1, 'R': 2, 'S': 1,
            'T': 1, 'V': 0, 'W': 1, 'Y': 1}
_ACCEPT_T = {'A': 0, 'C': 0, 'D': 2, 'E': 2, 'F': 0, 'G': 0, 'H': 1,
             'I': 0, 'K': 0, 'L': 0, 'M': 1, 'N': 1, 'P': 0, 'Q': 1,
             'R': 0, 'S': 1, 'T': 1, 'V': 0, 'W': 0, 'Y': 1}
_AA = "ACDEFGHIKLMNPQRSTVWY"


def _targets(B, S):
    hyd = np.array([(_HYDRO_T[a] + 5.0) / 10.0 for a in _AA], np.float32)
    cls = np.array([0 if _CHARGE_T[a] > 0 else (1 if _CHARGE_T[a] < 0 else 2)
                    for a in _AA], np.int32)
    don = np.array([_DONOR_T[a] for a in _AA], np.float32)
    acc = np.array([_ACCEPT_T[a] for a in _AA], np.float32)
    idx = (np.arange(B, dtype=np.int64)[:, None] * S
           + np.arange(S, dtype=np.int64)[None, :]) % 20
    t_hydro = hyd[idx][..., None]
    labels = cls[idx].reshape(-1)
    t_hbond = np.stack([don[idx], acc[idx]], axis=-1)
    return jnp.asarray(t_hydro), jnp.asarray(labels), jnp.asarray(t_hbond)


def _row_kernel(x_ref, w1_ref, b1_ref, w2_ref, b2_ref, w3_ref, b3_ref,
                pa_w_ref, pb_w_ref, pb_b_ref,
                props_ref, feat_ref, pa_ref, pb_ref):
    x = x_ref[...]
    xb = x.astype(_BF16)
    h1 = jnp.maximum(
        jnp.dot(xb, w1_ref[...], preferred_element_type=jnp.float32)
        + b1_ref[...], 0.0)
    h2 = jnp.maximum(
        jnp.dot(h1.astype(_BF16), w2_ref[...],
                preferred_element_type=jnp.float32) + b2_ref[...], 0.0)
    z = jnp.dot(h2.astype(_BF16), w3_ref[...],
                preferred_element_type=jnp.float32) + b3_ref[...]

    hydro = jax.nn.sigmoid(z[:, 0:1])
    cz = z[:, 1:4]
    ce = jnp.exp(cz - jnp.max(cz, axis=1, keepdims=True))
    cp = ce / jnp.sum(ce, axis=1, keepdims=True)
    hb = z[:, 4:6]
    props_ref[...] = jnp.concatenate([hydro, cp, hb, z[:, 6:8]], axis=1)

    mod = ((1.0 + _CW * (hydro - 0.5))
           * (1.0 + _CW * (cp[:, 0:1] - cp[:, 1:2]))
           * (1.0 + _CW * (0.5 * (hb[:, 0:1] + hb[:, 1:2]) - 0.5)))
    feat_ref[...] = x * mod

    pa_ref[...] = jnp.dot(x, pa_w_ref[...], preferred_element_type=jnp.float32)
    pb_ref[...] = (jnp.dot(x, pb_w_ref[...], preferred_element_type=jnp.float32)
                   + pb_b_ref[...])


def _pair_kernel(pa_ref, pb_ref, don_ref, acc_ref, st_ref, stt_ref, stn_ref,
                 w2_ref, b2_ref, w3_ref, b3_ref, sp_ref, hb_ref):
    ti, hdim = pa_ref.shape
    spn = pb_ref.shape[0]

    h1 = jnp.maximum(pa_ref[...][:, None, :] + pb_ref[...][None, :, :], 0.0)
    h1f = h1.reshape(ti * spn, hdim).astype(_BF16)
    h2 = jnp.maximum(
        jnp.dot(h1f, w2_ref[...], preferred_element_type=jnp.float32)
        + b2_ref[...], 0.0)
    zlin = jnp.dot(h2, w3_ref[...],
                   preferred_element_type=jnp.float32) + b3_ref[0, 0]
    p = jax.nn.sigmoid(zlin.reshape(ti, spn))

    si = st_ref[...]
    si_n = jnp.sum(si * si, axis=1, keepdims=True)
    dots = jnp.dot(si, stt_ref[...], preferred_element_type=jnp.float32)
    d2 = si_n + stn_ref[...] - 2.0 * dots

    contact = d2 < 64.0
    hmask = jnp.logical_and(d2 > 6.25, d2 < 12.25)

    log_p = jnp.maximum(jnp.log(p), -100.0)
    log_1mp = jnp.maximum(jnp.log(1.0 - p), -100.0)
    term = jnp.where(contact, -log_p, -log_1mp)
    rows = (pl.program_id(1) * ti
            + jax.lax.broadcasted_iota(jnp.int32, (ti, spn), 0))
    cols = jax.lax.broadcasted_iota(jnp.int32, (ti, spn), 1)
    sp_sum = jnp.sum(jnp.where(cols > rows, 2.0 * term, 0.0))

    lgt = don_ref[...] * acc_ref[...]
    y = hmask.astype(jnp.float32)
    hb_sum = jnp.sum(jnp.maximum(lgt, 0.0) - lgt * y
                     + jnp.log1p(jnp.exp(-jnp.abs(lgt))))

    sp_ref[...] = jnp.broadcast_to(sp_sum, (1, 8, 128))
    hb_ref[...] = jnp.broadcast_to(hb_sum, (1, 8, 128))


def kernel(features, structures, w1, b1, w2, b2, w3, b3,
           sw1_top, sw1_bot, sb1, sw2t, sb2c, sw3c, sb3):
    B, S, D = features.shape
    H = sw1_top.shape[1]
    N = B * S
    x2d = features.reshape(N, D).astype(jnp.float32)

    tr = _ROW_TILE
    grid1 = (N // tr,)
    w_full = lambda w: pl.BlockSpec(w.shape, lambda i: (0,) * w.ndim)
    in_specs1 = ([pl.BlockSpec((tr, D), lambda i: (i, 0))]
                 + [w_full(w) for w in (w1, b1, w2, b2, w3, b3,
                                        sw1_top, sw1_bot, sb1)])
    out_specs1 = [pl.BlockSpec((tr, 8), lambda i: (i, 0)),
                  pl.BlockSpec((tr, D), lambda i: (i, 0)),
                  pl.BlockSpec((tr, H), lambda i: (i, 0)),
                  pl.BlockSpec((tr, H), lambda i: (i, 0))]
    out_shape1 = (jax.ShapeDtypeStruct((N, 8), jnp.float32),
                  jax.ShapeDtypeStruct((N, D), jnp.float32),
                  jax.ShapeDtypeStruct((N, H), jnp.float32),
                  jax.ShapeDtypeStruct((N, H), jnp.float32))
    props, feat2d, pa, pb = pl.pallas_call(
        _row_kernel, grid=grid1, in_specs=in_specs1, out_specs=out_specs1,
        out_shape=out_shape1,
        compiler_params=pltpu.CompilerParams(
            dimension_semantics=("parallel",),
            vmem_limit_bytes=_VMEM_LIMIT),
    )(x2d, w1, b1, w2, b2, w3, b3, sw1_top, sw1_bot, sb1)

    constrained = feat2d.reshape(B, S, D)
    props3 = props.reshape(B, S, 8)
    hydro = props3[:, :, 0:1]
    charge = props[:, 1:4]
    hbond = props3[:, :, 4:6]

    ti = _PAIR_TI
    pa3 = pa.reshape(B, S, H)
    pb3 = pb.reshape(B, S, H)
    don = props3[:, :, 4:5]
    acc = props3[:, :, 5][:, None, :]
    st = jnp.pad(structures.astype(jnp.float32), ((0, 0), (0, 0), (0, 5)))
    stt = jnp.swapaxes(st, 1, 2)
    stn = jnp.sum(st * st, axis=2)[:, None, :]
    w2p = sw2t.T
    b2r = sb2c.T

    grid2 = (B, S // ti)
    in_specs2 = [
        pl.BlockSpec((None, ti, H), lambda b, i: (b, i, 0)),
        pl.BlockSpec((None, S, H), lambda b, i: (b, 0, 0)),
        pl.BlockSpec((None, ti, 1), lambda b, i: (b, i, 0)),
        pl.BlockSpec((None, 1, S), lambda b, i: (b, 0, 0)),
        pl.BlockSpec((None, ti, 8), lambda b, i: (b, i, 0)),
        pl.BlockSpec((None, 8, S), lambda b, i: (b, 0, 0)),
        pl.BlockSpec((None, 1, S), lambda b, i: (b, 0, 0)),
        pl.BlockSpec(w2p.shape, lambda b, i: (0, 0)),
        pl.BlockSpec(b2r.shape, lambda b, i: (0, 0)),
        pl.BlockSpec(sw3c.shape, lambda b, i: (0, 0)),
        pl.BlockSpec(sb3.shape, lambda b, i: (0, 0)),
    ]
    out_specs2 = [pl.BlockSpec((None, 1, 8, 128), lambda b, i: (b, i, 0, 0)),
                  pl.BlockSpec((None, 1, 8, 128), lambda b, i: (b, i, 0, 0))]
    out_shape2 = (jax.ShapeDtypeStruct((B, S // ti, 8, 128), jnp.float32),
                  jax.ShapeDtypeStruct((B, S // ti, 8, 128), jnp.float32))
    sp_parts, hb_parts = pl.pallas_call(
        _pair_kernel, grid=grid2, in_specs=in_specs2, out_specs=out_specs2,
        out_shape=out_shape2,
        compiler_params=pltpu.CompilerParams(
            dimension_semantics=("parallel", "parallel"),
            vmem_limit_bytes=_VMEM_LIMIT),
    )(pa3, pb3, don, acc, st, stt, stn, w2p, b2r, sw3c, sb3)

    t_hydro, labels, t_hbond = _targets(B, S)
    losses = {}
    losses['hydrophobic'] = jnp.mean((hydro.reshape(B, S, 1) - t_hydro) ** 2)

    lse = jax.scipy.special.logsumexp(charge, axis=-1)
    picked = jnp.take_along_axis(charge, labels[:, None], axis=-1)[:, 0]
    losses['charge'] = jnp.mean(lse - picked)

    n_pairs = float(B * S * S)
    hb_bce = jnp.sum(hb_parts[:, :, 0, 0]) / n_pairs
    losses['hbond'] = jnp.mean((hbond - t_hbond) ** 2) + 0.5 * hb_bce

    losses['spatial'] = (jnp.sum(sp_parts[:, :, 0, 0]) + 100.0 * B * S) / n_pairs

    return constrained, losses


# speedup vs baseline: 1.4871x; 1.0780x over previous
import numpy as np

import jax
import jax.numpy as jnp
from jax.experimental import pallas as pl
from jax.experimental.pallas import tpu as pltpu

_BF16 = jnp.bfloat16
_VMEM_LIMIT = 48 * 1024 * 1024
_CW = 0.1
_ROW_TILE = 512
_PAIR_TI = 64

_HYDRO_T = {'A': 1.8, 'C': 2.5, 'D': -3.5, 'E': -3.5, 'F': 2.8, 'G': -0.4,
            'H': -3.2, 'I': 4.5, 'K': -3.9, 'L': 3.8, 'M': 1.9, 'N': -3.5,
            'P': -1.6, 'Q': -3.5, 'R': -4.5, 'S': -0.8, 'T': -0.7,
            'V': 4.2, 'W': -0.9, 'Y': -1.3}
_CHARGE_T = {'A': 0, 'C': 0, 'D': -1, 'E': -1, 'F': 0, 'G': 0, 'H': 0.5,
             'I': 0, 'K': 1, 'L': 0, 'M': 0, 'N': 0, 'P': 0, 'Q': 0,
             'R': 1, 'S': 0, 'T': 0, 'V': 0, 'W': 0, 'Y': 0}
_DONOR_T = {'A': 0, 'C': 0, 'D': 0, 'E': 0, 'F': 0, 'G': 0, 'H': 1, 'I': 0,
            'K': 1, 'L': 0, 'M': 0, 'N': 1, 'P': 0, 'Q': 1, 'R': 2, 'S': 1,
            'T': 1, 'V': 0, 'W': 1, 'Y': 1}
_ACCEPT_T = {'A': 0, 'C': 0, 'D': 2, 'E': 2, 'F': 0, 'G': 0, 'H': 1,
             'I': 0, 'K': 0, 'L': 0, 'M': 1, 'N': 1, 'P': 0, 'Q': 1,
             'R': 0, 'S': 1, 'T': 1, 'V': 0, 'W': 0, 'Y': 1}
_AA = "ACDEFGHIKLMNPQRSTVWY"


def _targets(B, S):
    hyd = np.array([(_HYDRO_T[a] + 5.0) / 10.0 for a in _AA], np.float32)
    cls = np.array([0 if _CHARGE_T[a] > 0 else (1 if _CHARGE_T[a] < 0 else 2)
                    for a in _AA], np.int32)
    don = np.array([_DONOR_T[a] for a in _AA], np.float32)
    acc = np.array([_ACCEPT_T[a] for a in _AA], np.float32)
    idx = (np.arange(B, dtype=np.int64)[:, None] * S
           + np.arange(S, dtype=np.int64)[None, :]) % 20
    t_hydro = hyd[idx][..., None]
    labels = cls[idx].reshape(-1)
    t_hbond = np.stack([don[idx], acc[idx]], axis=-1)
    return jnp.asarray(t_hydro), jnp.asarray(labels), jnp.asarray(t_hbond)


def _row_kernel(x_ref, w1_ref, b1_ref, w2_ref, b2_ref, w3_ref, b3_ref,
                pa_w_ref, pb_w_ref, pb_b_ref,
                props_ref, feat_ref, pa_ref, pb_ref):
    x = x_ref[...]
    xb = x.astype(_BF16)
    h1 = jnp.maximum(
        jnp.dot(xb, w1_ref[...], preferred_element_type=jnp.float32)
        + b1_ref[...], 0.0)
    h2 = jnp.maximum(
        jnp.dot(h1.astype(_BF16), w2_ref[...],
                preferred_element_type=jnp.float32) + b2_ref[...], 0.0)
    z = jnp.dot(h2.astype(_BF16), w3_ref[...],
                preferred_element_type=jnp.float32) + b3_ref[...]

    hydro = jax.nn.sigmoid(z[:, 0:1])
    cz = z[:, 1:4]
    ce = jnp.exp(cz - jnp.max(cz, axis=1, keepdims=True))
    cp = ce / jnp.sum(ce, axis=1, keepdims=True)
    hb = z[:, 4:6]
    props_ref[...] = jnp.concatenate([hydro, cp, hb, z[:, 6:8]], axis=1)

    mod = ((1.0 + _CW * (hydro - 0.5))
           * (1.0 + _CW * (cp[:, 0:1] - cp[:, 1:2]))
           * (1.0 + _CW * (0.5 * (hb[:, 0:1] + hb[:, 1:2]) - 0.5)))
    feat_ref[...] = x * mod

    pa_ref[...] = jnp.dot(x, pa_w_ref[...], preferred_element_type=jnp.float32)
    pb_ref[...] = (jnp.dot(x, pb_w_ref[...], preferred_element_type=jnp.float32)
                   + pb_b_ref[...])


def _pair_kernel(pa_ref, pb_ref, don_ref, acc_ref, st_ref, stt_ref, stn_ref,
                 w2_ref, b2_ref, w3_ref, b3_ref, sp_ref, hb_ref):
    ti, hdim = pa_ref.shape
    spn = pb_ref.shape[0]

    h1 = jnp.maximum(pa_ref[...][:, None, :] + pb_ref[...][None, :, :], 0.0)
    h1f = h1.reshape(ti * spn, hdim).astype(_BF16)
    h2 = jnp.maximum(
        jnp.dot(h1f, w2_ref[...], preferred_element_type=jnp.float32)
        + b2_ref[...], 0.0)
    zlin = jnp.dot(h2, w3_ref[...],
                   preferred_element_type=jnp.float32) + b3_ref[0, 0]
    p = jax.nn.sigmoid(zlin.reshape(ti, spn))

    si = st_ref[...]
    si_n = jnp.sum(si * si, axis=1, keepdims=True)
    dots = jnp.dot(si, stt_ref[...], preferred_element_type=jnp.float32)
    d2 = si_n + stn_ref[...] - 2.0 * dots

    contact = d2 < 64.0
    hmask = jnp.logical_and(d2 > 6.25, d2 < 12.25)

    log_p = jnp.maximum(jnp.log(p), -100.0)
    log_1mp = jnp.maximum(jnp.log(1.0 - p), -100.0)
    term = jnp.where(contact, -log_p, -log_1mp)
    rows = (pl.program_id(1) * ti
            + jax.lax.broadcasted_iota(jnp.int32, (ti, spn), 0))
    cols = jax.lax.broadcasted_iota(jnp.int32, (ti, spn), 1)
    sp_sum = jnp.sum(jnp.where(cols > rows, 2.0 * term, 0.0))

    lgt = don_ref[...] * acc_ref[...]
    y = hmask.astype(jnp.float32)
    hb_sum = jnp.sum(jnp.maximum(lgt, 0.0) - lgt * y
                     + jnp.log1p(jnp.exp(-jnp.abs(lgt))))

    sp_ref[...] = jnp.broadcast_to(sp_sum, (1, 8, 128))
    hb_ref[...] = jnp.broadcast_to(hb_sum, (1, 8, 128))


def kernel(features, structures, w1, b1, w2, b2, w3, b3,
           sw1_top, sw1_bot, sb1, sw2t, sb2c, sw3c, sb3):
    B, S, D = features.shape
    H = sw1_top.shape[1]
    N = B * S
    x2d = features.reshape(N, D).astype(jnp.float32)

    tr = _ROW_TILE
    grid1 = (N // tr,)
    w_full = lambda w: pl.BlockSpec(w.shape, lambda i: (0,) * w.ndim)
    in_specs1 = ([pl.BlockSpec((tr, D), lambda i: (i, 0))]
                 + [w_full(w) for w in (w1, b1, w2, b2, w3, b3,
                                        sw1_top, sw1_bot, sb1)])
    out_specs1 = [pl.BlockSpec((tr, 8), lambda i: (i, 0)),
                  pl.BlockSpec((tr, D), lambda i: (i, 0)),
                  pl.BlockSpec((tr, H), lambda i: (i, 0)),
                  pl.BlockSpec((tr, H), lambda i: (i, 0))]
    out_shape1 = (jax.ShapeDtypeStruct((N, 8), jnp.float32),
                  jax.ShapeDtypeStruct((N, D), jnp.float32),
                  jax.ShapeDtypeStruct((N, H), jnp.float32),
                  jax.ShapeDtypeStruct((N, H), jnp.float32))
    props, feat2d, pa, pb = pl.pallas_call(
        _row_kernel, grid=grid1, in_specs=in_specs1, out_specs=out_specs1,
        out_shape=out_shape1,
        compiler_params=pltpu.CompilerParams(
            dimension_semantics=("parallel",),
            vmem_limit_bytes=_VMEM_LIMIT),
    )(x2d, w1, b1, w2, b2, w3, b3, sw1_top, sw1_bot, sb1)

    constrained = feat2d.reshape(B, S, D)
    props3 = props.reshape(B, S, 8)
    hydro = props3[:, :, 0:1]
    charge = props[:, 1:4]
    hbond = props3[:, :, 4:6]

    ti = _PAIR_TI
    pa3 = pa.reshape(B, S, H)
    pb3 = pb.reshape(B, S, H)
    don = props3[:, :, 4:5]
    acc = props3[:, :, 5][:, None, :]
    st = jnp.pad(structures.astype(jnp.float32), ((0, 0), (0, 0), (0, 5)))
    stt = jnp.swapaxes(st, 1, 2)
    stn = jnp.sum(st * st, axis=2)[:, None, :]
    w2p = sw2t.T
    b2r = sb2c.T

    grid2 = (B, S // ti)
    in_specs2 = [
        pl.BlockSpec((None, ti, H), lambda b, i: (b, i, 0)),
        pl.BlockSpec((None, S, H), lambda b, i: (b, 0, 0)),
        pl.BlockSpec((None, ti, 1), lambda b, i: (b, i, 0)),
        pl.BlockSpec((None, 1, S), lambda b, i: (b, 0, 0)),
        pl.BlockSpec((None, ti, 8), lambda b, i: (b, i, 0)),
        pl.BlockSpec((None, 8, S), lambda b, i: (b, 0, 0)),
        pl.BlockSpec((None, 1, S), lambda b, i: (b, 0, 0)),
        pl.BlockSpec(w2p.shape, lambda b, i: (0, 0)),
        pl.BlockSpec(b2r.shape, lambda b, i: (0, 0)),
        pl.BlockSpec(sw3c.shape, lambda b, i: (0, 0)),
        pl.BlockSpec(sb3.shape, lambda b, i: (0, 0)),
    ]
    out_specs2 = [pl.BlockSpec((None, 1, 8, 128), lambda b, i: (b, i, 0, 0)),
                  pl.BlockSpec((None, 1, 8, 128), lambda b, i: (b, i, 0, 0))]
    out_shape2 = (jax.ShapeDtypeStruct((B, S // ti, 8, 128), jnp.float32),
                  jax.ShapeDtypeStruct((B, S // ti, 8, 128), jnp.float32))
    sp_parts, hb_parts = pl.pallas_call(
        _pair_kernel, grid=grid2, in_specs=in_specs2, out_specs=out_specs2,
        out_shape=out_shape2,
        compiler_params=pltpu.CompilerParams(
            dimension_semantics=("parallel", "parallel"),
            vmem_limit_bytes=_VMEM_LIMIT),
    )(pa3, pb3, don, acc, st, stt, stn, w2p, b2r, sw3c, sb3)

    t_hydro, labels, t_hbond = _targets(B, S)
    losses = {}
    losses['hydrophobic'] = jnp.mean((hydro.reshape(B, S, 1) - t_hydro) ** 2)

    lse = jax.scipy.special.logsumexp(charge, axis=-1)
    picked = jnp.take_along_axis(charge, labels[:, None], axis=-1)[:, 0]
    losses['charge'] = jnp.mean(lse - picked)

    n_pairs = float(B * S * S)
    hb_bce = jnp.sum(hb_parts[:, :, 0, 0]) / n_pairs
    losses['hbond'] = jnp.mean((hbond - t_hbond) ** 2) + 0.5 * hb_bce

    losses['spatial'] = (jnp.sum(sp_parts[:, :, 0, 0]) + 100.0 * B * S) / n_pairs

    return constrained, losses
